# Initial kernel scaffold; baseline (speedup 1.0000x reference)
import jax
import jax.numpy as jnp
from jax.experimental import pallas as pl


def kernel(x, grid):
    raise NotImplementedError("write your pallas kernel here")


# speedup vs baseline: 1.2256x; 1.2256x over previous
import functools

import jax
import jax.numpy as jnp
from jax import lax
from jax.experimental import pallas as pl
from jax.experimental.pallas import tpu as pltpu
from jax.experimental.pallas import tpu_sc as plsc

L = 16


def _floor_i32(v):
    i = v.astype(jnp.int32)
    return jnp.where(i.astype(jnp.float32) > v, i - 1, i)


def _make_sc_call(N, C, H, W, NC, NS, CHUNK):
    P = N * H * W
    NW = NC * NS
    PPW = P // NW
    NCHUNKS = PPW // CHUNK
    HWsz = H * W
    G16 = CHUNK // L

    mesh = plsc.VectorSubcoreMesh(
        core_axis_name="c", subcore_axis_name="s", num_cores=NC, num_subcores=NS
    )

    @functools.partial(
        pl.kernel,
        out_type=jax.ShapeDtypeStruct((P, C), jnp.float32),
        mesh=mesh,
        compiler_params=pltpu.CompilerParams(
            needs_layout_passes=False, use_tc_tiling_on_sc=False),
        scratch_types=[
            pltpu.VMEM((CHUNK,), jnp.float32),
            pltpu.VMEM((CHUNK,), jnp.float32),
            pltpu.VMEM((4, CHUNK), jnp.int32),
            pltpu.VMEM((4 * CHUNK,), jnp.float32),
            pltpu.VMEM((4, CHUNK, C), jnp.float32),
            pltpu.VMEM((CHUNK, C), jnp.float32),
            pltpu.SemaphoreType.DMA,
        ],
    )
    def sc_call(gx_hbm, gy_hbm, table_hbm, out_hbm,
                gx_v, gy_v, idx_v, w_v, rows_v, out_v, gsem):
        cid = lax.axis_index("c")
        sid = lax.axis_index("s")
        wid = sid * NC + cid
        wbase = wid * PPW

        def chunk_body(g, carry):
            base = wbase + g * CHUNK
            pltpu.sync_copy(gx_hbm.at[pl.ds(base, CHUNK)], gx_v)
            pltpu.sync_copy(gy_hbm.at[pl.ds(base, CHUNK)], gy_v)
            nbase = (base // HWsz) * HWsz

            def coord_body(t, c2):
                gx = gx_v[pl.ds(t * L, L)]
                gy = gy_v[pl.ds(t * L, L)]
                ix = (gx + 1.0) * (W * 0.5) - 0.5
                iy = (gy + 1.0) * (H * 0.5) - 0.5
                ix0 = _floor_i32(ix)
                iy0 = _floor_i32(iy)
                wx1 = ix - ix0.astype(jnp.float32)
                wx0 = 1.0 - wx1
                wy1 = iy - iy0.astype(jnp.float32)
                wy0 = 1.0 - wy1
                for k, (dy, dx, wy, wx) in enumerate(
                    ((0, 0, wy0, wx0), (0, 1, wy0, wx1),
                     (1, 0, wy1, wx0), (1, 1, wy1, wx1))):
                    xi = ix0 + dx
                    yi = iy0 + dy
                    valid = ((xi >= 0) & (xi <= W - 1)
                             & (yi >= 0) & (yi <= H - 1))
                    xc = jnp.maximum(jnp.minimum(xi, W - 1), 0)
                    yc = jnp.maximum(jnp.minimum(yi, H - 1), 0)
                    idx_v[k, pl.ds(t * L, L)] = nbase + yc * W + xc
                    w_v[pl.ds(k * CHUNK + t * L, L)] = jnp.where(valid, wy * wx, 0.0)
                return c2

            lax.fori_loop(0, G16, coord_body, 0, unroll=False)

            copies = [
                pltpu.async_copy(table_hbm.at[idx_v.at[k]], rows_v.at[k], gsem)
                for k in range(4)
            ]
            for cp in copies:
                cp.wait()

            def point_body(t, c2):
                ws = [
                    plsc.load_gather(
                        w_v, [jnp.full((L,), k * CHUNK + t, jnp.int32)])
                    for k in range(4)
                ]
                for j in range(C // L):
                    acc = ws[0] * rows_v[0, t, pl.ds(j * L, L)]
                    acc += ws[1] * rows_v[1, t, pl.ds(j * L, L)]
                    acc += ws[2] * rows_v[2, t, pl.ds(j * L, L)]
                    acc += ws[3] * rows_v[3, t, pl.ds(j * L, L)]
                    out_v[t, pl.ds(j * L, L)] = acc
                return c2

            lax.fori_loop(0, CHUNK, point_body, 0, unroll=False)

            pltpu.sync_copy(out_v, out_hbm.at[pl.ds(base, CHUNK)])
            return carry

        lax.fori_loop(0, NCHUNKS, chunk_body, 0, unroll=False)

    return sc_call


@jax.jit
def kernel(x, grid):
    N, C, H, W = x.shape
    P = N * H * W
    table = jnp.transpose(x, (0, 2, 3, 1)).reshape(P, C)
    gx = grid[..., 0].reshape(P)
    gy = grid[..., 1].reshape(P)
    sc_call = _make_sc_call(N, C, H, W, NC=2, NS=16, CHUNK=128)
    out = sc_call(gx, gy, table)
    return jnp.transpose(out.reshape(N, H, W, C), (0, 3, 1, 2))
